# baseline (speedup 1.0000x reference)
import functools

import jax
import jax.numpy as jnp
from jax import lax
from jax.experimental import pallas as pl
from jax.experimental.pallas import tpu as pltpu
from jax.experimental.pallas import tpu_sc as plsc

VOCAB = 1000000
D = 64
B = 4096
L = 200
PAD = 0

NC = 2
NS = 16
NW = NC * NS
BT = 128
DT = 8


def _gather_body(tokT_hbm, table_hbm, out_hbm, idx_v, rows_v, stage_v,
                 isems, gsems, wsems):
    w = lax.axis_index("s") * NC + lax.axis_index("c")
    b0 = w * BT
    iota16 = lax.iota(jnp.int32, 16)

    def idx_copy(l, par):
        pltpu.sync_copy(tokT_hbm.at[l, pl.ds(b0, BT)], idx_v[par])

    def start_gather(par):
        pltpu.async_copy(table_hbm.at[idx_v[par]], rows_v[par], gsems[par])

    def wait_gather(par):
        pltpu.make_async_copy(
            table_hbm.at[idx_v[par]], rows_v[par], gsems[par]
        ).wait()

    def start_write(l, par):
        pltpu.async_copy(stage_v[par], out_hbm.at[l, :, w], wsems[par])

    def wait_write(par):
        pltpu.make_async_copy(
            stage_v[par], out_hbm.at[0, :, 0], wsems[par]
        ).wait()

    def transpose(par):
        rows = rows_v[par]
        stage = stage_v[par]

        def dt_step(dt, carry):
            for s in range(8):
                d = dt * 8 + s
                col = jnp.full((16,), d, jnp.int32)
                for k in range(8):
                    v = plsc.load_gather(rows, [iota16 + 16 * k, col])
                    stage[dt, s, pl.ds(16 * k, 16)] = v
            return carry

        lax.fori_loop(0, DT, dt_step, 0)

    idx_copy(0, 0)
    start_gather(0)
    idx_copy(1, 1)

    def body(gg, carry):
        for par in range(2):
            l = 2 * gg + par
            wait_gather(par)
            if par == 0:
                start_gather(1)
            else:
                @pl.when(gg < L // 2 - 1)
                def _():
                    start_gather(0)

            @pl.when(gg >= 1)
            def _():
                wait_write(par)

            transpose(par)
            start_write(l, par)

            @pl.when(gg < L // 2 - 1)
            def _():
                idx_copy(l + 2, par)

        return carry

    lax.fori_loop(0, L // 2, body, 0)
    wait_write(0)
    wait_write(1)


def _sc_gather(tokens_T, table):
    mesh = plsc.VectorSubcoreMesh(core_axis_name="c", subcore_axis_name="s")
    k = functools.partial(
        pl.kernel,
        mesh=mesh,
        out_type=jax.ShapeDtypeStruct((L, DT, NW, 8, BT), jnp.float32),
        scratch_types=[
            [pltpu.VMEM((BT,), jnp.int32) for _ in range(2)],
            [pltpu.VMEM((BT, D), jnp.float32) for _ in range(2)],
            [pltpu.VMEM((DT, 8, BT), jnp.float32) for _ in range(2)],
            [pltpu.SemaphoreType.DMA for _ in range(2)],
            [pltpu.SemaphoreType.DMA for _ in range(2)],
            [pltpu.SemaphoreType.DMA for _ in range(2)],
        ],
        compiler_params=pltpu.CompilerParams(
            use_tc_tiling_on_sc=False, needs_layout_passes=False
        ),
    )(_gather_body)
    return k(tokens_T, table)


def _mask_body(tok_ref, pad_ref, seq_ref):
    pad_ref[...] = (tok_ref[...] == PAD).astype(jnp.int8)
    r = lax.broadcasted_iota(jnp.int32, (L, L), 0)
    c = lax.broadcasted_iota(jnp.int32, (L, L), 1)
    seq_ref[...] = (c > r).astype(jnp.int8)


def _tc_masks(tokens):
    return pl.pallas_call(
        _mask_body,
        out_shape=(
            jax.ShapeDtypeStruct((B, L), jnp.int8),
            jax.ShapeDtypeStruct((L, L), jnp.int8),
        ),
    )(tokens)


def kernel(tokens, table):
    arr = _sc_gather(tokens.T, table)
    features = arr.transpose(2, 4, 0, 1, 3).reshape(B, L, D)
    pad8, seq8 = _tc_masks(tokens)
    padding_masks = pad8.astype(bool)[:, None, None, :]
    sequential_masks = seq8.astype(bool)
    return (features, padding_masks, sequential_masks)


# speedup vs baseline: 1.1432x; 1.1432x over previous
import functools

import jax
import jax.numpy as jnp
from jax import lax
from jax.experimental import pallas as pl
from jax.experimental.pallas import tpu as pltpu
from jax.experimental.pallas import tpu_sc as plsc

VOCAB = 1000000
D = 64
B = 4096
L = 200
PAD = 0

NC = 2
NS = 16
NW = NC * NS
BT = 128
DT = 8


def _gather_body(tokT_hbm, table_hbm, out_hbm, idx_v, rows_v, stage_v,
                 isems, gsems, wsems):
    w = lax.axis_index("s") * NC + lax.axis_index("c")
    b0 = w * BT
    iota16 = lax.iota(jnp.int32, 16)

    def idx_copy(l, par):
        pltpu.sync_copy(tokT_hbm.at[l, pl.ds(b0, BT)], idx_v[par])

    def start_gather(par):
        pltpu.async_copy(table_hbm.at[idx_v[par]], rows_v[par], gsems[par])

    def wait_gather(par):
        pltpu.make_async_copy(
            table_hbm.at[idx_v[par]], rows_v[par], gsems[par]
        ).wait()

    def start_write(l, par):
        pltpu.async_copy(stage_v[par], out_hbm.at[l, :, w], wsems[par])

    def wait_write(par):
        pltpu.make_async_copy(
            stage_v[par], out_hbm.at[0, :, 0], wsems[par]
        ).wait()

    def transpose(par):
        rows = rows_v[par]
        stage = stage_v[par]
        for dt in range(8):
            for s in range(8):
                d = dt * 8 + s
                col = jnp.full((16,), d, jnp.int32)
                vs = [
                    plsc.load_gather(rows, [iota16 + 16 * k, col])
                    for k in range(8)
                ]
                for k in range(8):
                    stage[dt, s, pl.ds(16 * k, 16)] = vs[k]

    idx_copy(0, 0)
    start_gather(0)
    idx_copy(1, 1)

    def body(gg, carry):
        for par in range(2):
            l = 2 * gg + par
            wait_gather(par)
            if par == 0:
                start_gather(1)
            else:
                @pl.when(gg < L // 2 - 1)
                def _():
                    start_gather(0)

            @pl.when(gg >= 1)
            def _():
                wait_write(par)

            transpose(par)
            start_write(l, par)

            @pl.when(gg < L // 2 - 1)
            def _():
                idx_copy(l + 2, par)

        return carry

    lax.fori_loop(0, L // 2, body, 0)
    wait_write(0)
    wait_write(1)


def _sc_gather(tokens_T, table):
    mesh = plsc.VectorSubcoreMesh(core_axis_name="c", subcore_axis_name="s")
    k = functools.partial(
        pl.kernel,
        mesh=mesh,
        out_type=jax.ShapeDtypeStruct((L, DT, NW, 8, BT), jnp.float32),
        scratch_types=[
            [pltpu.VMEM((BT,), jnp.int32) for _ in range(2)],
            [pltpu.VMEM((BT, D), jnp.float32) for _ in range(2)],
            [pltpu.VMEM((DT, 8, BT), jnp.float32) for _ in range(2)],
            [pltpu.SemaphoreType.DMA for _ in range(2)],
            [pltpu.SemaphoreType.DMA for _ in range(2)],
            [pltpu.SemaphoreType.DMA for _ in range(2)],
        ],
        compiler_params=pltpu.CompilerParams(
            use_tc_tiling_on_sc=False, needs_layout_passes=False
        ),
    )(_gather_body)
    return k(tokens_T, table)


def _mask_body(tok_ref, pad_ref, seq_ref):
    pad_ref[...] = (tok_ref[...] == PAD).astype(jnp.int8)
    r = lax.broadcasted_iota(jnp.int32, (L, L), 0)
    c = lax.broadcasted_iota(jnp.int32, (L, L), 1)
    seq_ref[...] = (c > r).astype(jnp.int8)


def _tc_masks(tokens):
    return pl.pallas_call(
        _mask_body,
        out_shape=(
            jax.ShapeDtypeStruct((B, L), jnp.int8),
            jax.ShapeDtypeStruct((L, L), jnp.int8),
        ),
    )(tokens)


def kernel(tokens, table):
    arr = _sc_gather(tokens.T, table)
    features = arr.transpose(2, 4, 0, 1, 3).reshape(B, L, D)
    pad8, seq8 = _tc_masks(tokens)
    padding_masks = pad8.astype(bool)[:, None, None, :]
    sequential_masks = seq8.astype(bool)
    return (features, padding_masks, sequential_masks)


# speedup vs baseline: 1.1481x; 1.0043x over previous
import functools

import jax
import jax.numpy as jnp
from jax import lax
from jax.experimental import pallas as pl
from jax.experimental.pallas import tpu as pltpu
from jax.experimental.pallas import tpu_sc as plsc

VOCAB = 1000000
D = 64
B = 4096
L = 200
PAD = 0

NC = 2
NS = 16
NW = NC * NS
BT = 128
DT = 8


NBUF = 4
GROUPS = L // NBUF


def _gather_body(tokT_hbm, table_hbm, out_hbm, idx_v, rows_v, stage_v,
                 isems, gsems, wsems):
    w = lax.axis_index("s") * NC + lax.axis_index("c")
    b0 = w * BT
    iota16 = lax.iota(jnp.int32, 16)

    def idx_copy_async(l, par):
        pltpu.async_copy(tokT_hbm.at[l, pl.ds(b0, BT)], idx_v[par], isems[par])

    def wait_idx(par):
        pltpu.make_async_copy(
            tokT_hbm.at[0, pl.ds(b0, BT)], idx_v[par], isems[par]
        ).wait()

    def start_gather(par):
        pltpu.async_copy(table_hbm.at[idx_v[par]], rows_v[par], gsems[par])

    def wait_gather(par):
        pltpu.make_async_copy(
            table_hbm.at[idx_v[par]], rows_v[par], gsems[par]
        ).wait()

    def start_write(l, par):
        pltpu.async_copy(stage_v[par], out_hbm.at[l, :, w], wsems[par])

    def wait_write(par):
        pltpu.make_async_copy(
            stage_v[par], out_hbm.at[0, :, 0], wsems[par]
        ).wait()

    def transpose(par):
        rows = rows_v[par]
        stage = stage_v[par]

        def dt_step(dt, carry):
            for s in range(8):
                d = dt * 8 + s
                col = jnp.full((16,), d, jnp.int32)
                vs = [
                    plsc.load_gather(rows, [iota16 + 16 * k, col])
                    for k in range(8)
                ]
                for k in range(8):
                    stage[dt, s, pl.ds(16 * k, 16)] = vs[k]
            return carry

        lax.fori_loop(0, DT, dt_step, 0)

    for par in range(NBUF):
        idx_copy_async(par, par)
    for par in range(NBUF):
        wait_idx(par)
        start_gather(par)

    def body(gg, carry):
        for par in range(NBUF):
            wait_gather(par)

            @pl.when(gg < GROUPS - 1)
            def _():
                idx_copy_async(NBUF * gg + NBUF + par, par)

        for par in range(NBUF):
            l = NBUF * gg + par

            @pl.when(gg >= 1)
            def _():
                wait_write(par)

            transpose(par)
            start_write(l, par)

        for par in range(NBUF):
            @pl.when(gg < GROUPS - 1)
            def _():
                wait_idx(par)
                start_gather(par)

        return carry

    lax.fori_loop(0, GROUPS, body, 0)
    for par in range(NBUF):
        wait_write(par)


def _sc_gather(tokens_T, table):
    mesh = plsc.VectorSubcoreMesh(core_axis_name="c", subcore_axis_name="s")
    k = functools.partial(
        pl.kernel,
        mesh=mesh,
        out_type=jax.ShapeDtypeStruct((L, DT, NW, 8, BT), jnp.float32),
        scratch_types=[
            [pltpu.VMEM((BT,), jnp.int32) for _ in range(NBUF)],
            [pltpu.VMEM((BT, D), jnp.float32) for _ in range(NBUF)],
            [pltpu.VMEM((DT, 8, BT), jnp.float32) for _ in range(NBUF)],
            [pltpu.SemaphoreType.DMA for _ in range(NBUF)],
            [pltpu.SemaphoreType.DMA for _ in range(NBUF)],
            [pltpu.SemaphoreType.DMA for _ in range(NBUF)],
        ],
        compiler_params=pltpu.CompilerParams(
            use_tc_tiling_on_sc=False, needs_layout_passes=False
        ),
    )(_gather_body)
    return k(tokens_T, table)


def _mask_body(tok_ref, pad_ref, seq_ref):
    pad_ref[...] = (tok_ref[...] == PAD).astype(jnp.int8)
    r = lax.broadcasted_iota(jnp.int32, (L, L), 0)
    c = lax.broadcasted_iota(jnp.int32, (L, L), 1)
    seq_ref[...] = (c > r).astype(jnp.int8)


def _tc_masks(tokens):
    return pl.pallas_call(
        _mask_body,
        out_shape=(
            jax.ShapeDtypeStruct((B, L), jnp.int8),
            jax.ShapeDtypeStruct((L, L), jnp.int8),
        ),
    )(tokens)


def kernel(tokens, table):
    arr = _sc_gather(tokens.T, table)
    features = arr.transpose(2, 4, 0, 1, 3).reshape(B, L, D)
    pad8, seq8 = _tc_masks(tokens)
    padding_masks = pad8.astype(bool)[:, None, None, :]
    sequential_masks = seq8.astype(bool)
    return (features, padding_masks, sequential_masks)


# speedup vs baseline: 1.7502x; 1.5244x over previous
import functools

import jax
import jax.numpy as jnp
from jax import lax
from jax.experimental import pallas as pl
from jax.experimental.pallas import tpu as pltpu
from jax.experimental.pallas import tpu_sc as plsc

VOCAB = 1000000
D = 64
B = 4096
L = 200
PAD = 0

NC = 2
NS = 16
NW = NC * NS
BT = 128
DT = 8


NBUF = 4
GROUPS = L // NBUF


def _gather_body(tokT_hbm, table_hbm, out_hbm, idx_v, rows_v, stage_v,
                 isems, gsems, wsems):
    w = lax.axis_index("s") * NC + lax.axis_index("c")
    b0 = w * BT
    iota16 = lax.iota(jnp.int32, 16)

    def idx_copy_async(l, par):
        pltpu.async_copy(tokT_hbm.at[l, pl.ds(b0, BT)], idx_v[par], isems[par])

    def wait_idx(par):
        pltpu.make_async_copy(
            tokT_hbm.at[0, pl.ds(b0, BT)], idx_v[par], isems[par]
        ).wait()

    def start_gather(par):
        pltpu.async_copy(table_hbm.at[idx_v[par]], rows_v[par], gsems[par])

    def wait_gather(par):
        pltpu.make_async_copy(
            table_hbm.at[idx_v[par]], rows_v[par], gsems[par]
        ).wait()

    def start_write(l, par):
        pltpu.async_copy(
            stage_v[par].at[:, :, pl.ds(0, BT)], out_hbm.at[l, :, w], wsems[par]
        )

    def wait_write(par):
        pltpu.make_async_copy(
            stage_v[par].at[:, :, pl.ds(0, BT)], out_hbm.at[0, :, 0], wsems[par]
        ).wait()

    dt_idx = [(16 * q + iota16) // 8 for q in range(D // 16)]
    s_idx = [(16 * q + iota16) % 8 for q in range(D // 16)]

    def transpose(par):
        rows = rows_v[par]
        stage = stage_v[par]

        def c_step(c, carry):
            bl = jnp.full((16,), 0, jnp.int32) + c
            for q in range(D // 16):
                v = rows[c, pl.ds(16 * q, 16)]
                plsc.store_scatter(stage, [dt_idx[q], s_idx[q], bl], v)
            return carry

        lax.fori_loop(0, BT, c_step, 0)

    for par in range(NBUF):
        idx_copy_async(par, par)
    for par in range(NBUF):
        wait_idx(par)
        start_gather(par)

    def body(gg, carry):
        for par in range(NBUF):
            wait_gather(par)

            @pl.when(gg < GROUPS - 1)
            def _():
                idx_copy_async(NBUF * gg + NBUF + par, par)

        for par in range(NBUF):
            l = NBUF * gg + par

            @pl.when(gg >= 1)
            def _():
                wait_write(par)

            transpose(par)
            start_write(l, par)

        for par in range(NBUF):
            @pl.when(gg < GROUPS - 1)
            def _():
                wait_idx(par)
                start_gather(par)

        return carry

    lax.fori_loop(0, GROUPS, body, 0)
    for par in range(NBUF):
        wait_write(par)


def _sc_gather(tokens_T, table):
    mesh = plsc.VectorSubcoreMesh(core_axis_name="c", subcore_axis_name="s")
    k = functools.partial(
        pl.kernel,
        mesh=mesh,
        out_type=jax.ShapeDtypeStruct((L, DT, NW, 8, BT), jnp.float32),
        scratch_types=[
            [pltpu.VMEM((BT,), jnp.int32) for _ in range(NBUF)],
            [pltpu.VMEM((BT, D), jnp.float32) for _ in range(NBUF)],
            [pltpu.VMEM((DT, 8, BT + 5), jnp.float32) for _ in range(NBUF)],
            [pltpu.SemaphoreType.DMA for _ in range(NBUF)],
            [pltpu.SemaphoreType.DMA for _ in range(NBUF)],
            [pltpu.SemaphoreType.DMA for _ in range(NBUF)],
        ],
        compiler_params=pltpu.CompilerParams(
            use_tc_tiling_on_sc=False, needs_layout_passes=False
        ),
    )(_gather_body)
    return k(tokens_T, table)


def _mask_body(tok_ref, pad_ref, seq_ref):
    pad_ref[...] = (tok_ref[...] == PAD).astype(jnp.int8)
    r = lax.broadcasted_iota(jnp.int32, (L, L), 0)
    c = lax.broadcasted_iota(jnp.int32, (L, L), 1)
    seq_ref[...] = (c > r).astype(jnp.int8)


def _tc_masks(tokens):
    return pl.pallas_call(
        _mask_body,
        out_shape=(
            jax.ShapeDtypeStruct((B, L), jnp.int8),
            jax.ShapeDtypeStruct((L, L), jnp.int8),
        ),
    )(tokens)


def kernel(tokens, table):
    arr = _sc_gather(tokens.T, table)
    features = arr.transpose(2, 4, 0, 1, 3).reshape(B, L, D)
    pad8, seq8 = _tc_masks(tokens)
    padding_masks = pad8.astype(bool)[:, None, None, :]
    sequential_masks = seq8.astype(bool)
    return (features, padding_masks, sequential_masks)


# speedup vs baseline: 2.0020x; 1.1439x over previous
import functools

import jax
import jax.numpy as jnp
from jax import lax
from jax.experimental import pallas as pl
from jax.experimental.pallas import tpu as pltpu
from jax.experimental.pallas import tpu_sc as plsc

VOCAB = 1000000
D = 64
B = 4096
L = 200
PAD = 0

NC = 2
NS = 16
NW = NC * NS
BT = 128
DT = 8


NBUF = 4
GROUPS = L // NBUF


def _gather_body(tokT_hbm, table_hbm, out_hbm, idx_v, rows_v, stage_v,
                 isems, gsems, wsems):
    w = lax.axis_index("s") * NC + lax.axis_index("c")
    b0 = w * BT
    iota16 = lax.iota(jnp.int32, 16)

    def idx_copy_async(l, par):
        pltpu.async_copy(tokT_hbm.at[l, pl.ds(b0, BT)], idx_v[par], isems[par])

    def wait_idx(par):
        pltpu.make_async_copy(
            tokT_hbm.at[0, pl.ds(b0, BT)], idx_v[par], isems[par]
        ).wait()

    def start_gather(par):
        pltpu.async_copy(table_hbm.at[idx_v[par]], rows_v[par], gsems[par])

    def wait_gather(par):
        pltpu.make_async_copy(
            table_hbm.at[idx_v[par]], rows_v[par], gsems[par]
        ).wait()

    def start_write(l, par):
        pltpu.async_copy(
            stage_v[par].at[:, :, pl.ds(0, BT)], out_hbm.at[l, :, w], wsems[par]
        )

    def wait_write(par):
        pltpu.make_async_copy(
            stage_v[par].at[:, :, pl.ds(0, BT)], out_hbm.at[0, :, 0], wsems[par]
        ).wait()

    dt_idx = [(16 * q + iota16) // 8 for q in range(D // 16)]
    s_idx = [(16 * q + iota16) % 8 for q in range(D // 16)]

    def transpose(par):
        rows = rows_v[par]
        stage = stage_v[par]

        def c_step(cc, carry):
            c0 = cc * 4
            bls = [jnp.full((16,), 0, jnp.int32) + (c0 + j) for j in range(4)]
            vs = [
                rows[c0 + j, pl.ds(16 * q, 16)]
                for j in range(4)
                for q in range(D // 16)
            ]
            for j in range(4):
                for q in range(D // 16):
                    plsc.store_scatter(
                        stage, [dt_idx[q], s_idx[q], bls[j]], vs[j * (D // 16) + q]
                    )
            return carry

        lax.fori_loop(0, BT // 4, c_step, 0)

    for par in range(NBUF):
        idx_copy_async(par, par)
    for par in range(NBUF):
        wait_idx(par)
        start_gather(par)

    def body(gg, carry):
        for par in range(NBUF):
            wait_gather(par)

            @pl.when(gg < GROUPS - 1)
            def _():
                idx_copy_async(NBUF * gg + NBUF + par, par)

        for par in range(NBUF):
            l = NBUF * gg + par

            @pl.when(gg >= 1)
            def _():
                wait_write(par)

            transpose(par)
            start_write(l, par)

        for par in range(NBUF):
            @pl.when(gg < GROUPS - 1)
            def _():
                wait_idx(par)
                start_gather(par)

        return carry

    lax.fori_loop(0, GROUPS, body, 0)
    for par in range(NBUF):
        wait_write(par)


def _sc_gather(tokens_T, table):
    mesh = plsc.VectorSubcoreMesh(core_axis_name="c", subcore_axis_name="s")
    k = functools.partial(
        pl.kernel,
        mesh=mesh,
        out_type=jax.ShapeDtypeStruct((L, DT, NW, 8, BT), jnp.float32),
        scratch_types=[
            [pltpu.VMEM((BT,), jnp.int32) for _ in range(NBUF)],
            [pltpu.VMEM((BT, D), jnp.float32) for _ in range(NBUF)],
            [pltpu.VMEM((DT, 8, BT + 5), jnp.float32) for _ in range(NBUF)],
            [pltpu.SemaphoreType.DMA for _ in range(NBUF)],
            [pltpu.SemaphoreType.DMA for _ in range(NBUF)],
            [pltpu.SemaphoreType.DMA for _ in range(NBUF)],
        ],
        compiler_params=pltpu.CompilerParams(
            use_tc_tiling_on_sc=False, needs_layout_passes=False
        ),
    )(_gather_body)
    return k(tokens_T, table)


def _mask_body(tok_ref, pad_ref, seq_ref):
    pad_ref[...] = (tok_ref[...] == PAD).astype(jnp.int8)
    r = lax.broadcasted_iota(jnp.int32, (L, L), 0)
    c = lax.broadcasted_iota(jnp.int32, (L, L), 1)
    seq_ref[...] = (c > r).astype(jnp.int8)


def _tc_masks(tokens):
    return pl.pallas_call(
        _mask_body,
        out_shape=(
            jax.ShapeDtypeStruct((B, L), jnp.int8),
            jax.ShapeDtypeStruct((L, L), jnp.int8),
        ),
    )(tokens)


def kernel(tokens, table):
    arr = _sc_gather(tokens.T, table)
    features = arr.transpose(2, 4, 0, 1, 3).reshape(B, L, D)
    pad8, seq8 = _tc_masks(tokens)
    padding_masks = pad8.astype(bool)[:, None, None, :]
    sequential_masks = seq8.astype(bool)
    return (features, padding_masks, sequential_masks)


# speedup vs baseline: 2.0041x; 1.0011x over previous
import functools

import jax
import jax.numpy as jnp
from jax import lax
from jax.experimental import pallas as pl
from jax.experimental.pallas import tpu as pltpu
from jax.experimental.pallas import tpu_sc as plsc

VOCAB = 1000000
D = 64
B = 4096
L = 200
PAD = 0

NC = 2
NS = 16
NW = NC * NS
BT = 128
DT = 8


NBUF = 4
GROUPS = L // NBUF


def _gather_body(tokT_hbm, table_hbm, out_hbm, idx_v, rows_v, stage_v,
                 isems, gsems, wsems):
    w = lax.axis_index("s") * NC + lax.axis_index("c")
    b0 = w * BT
    iota16 = lax.iota(jnp.int32, 16)

    def idx_copy_async(l, par):
        pltpu.async_copy(tokT_hbm.at[l, pl.ds(b0, BT)], idx_v[par], isems[par])

    def wait_idx(par):
        pltpu.make_async_copy(
            tokT_hbm.at[0, pl.ds(b0, BT)], idx_v[par], isems[par]
        ).wait()

    def start_gather(par):
        pltpu.async_copy(table_hbm.at[idx_v[par]], rows_v[par], gsems[par])

    def wait_gather(par):
        pltpu.make_async_copy(
            table_hbm.at[idx_v[par]], rows_v[par], gsems[par]
        ).wait()

    def start_write(l, par):
        pltpu.async_copy(
            stage_v[par].at[:, :, pl.ds(0, BT)], out_hbm.at[l, :, w], wsems[par]
        )

    def wait_write(par):
        pltpu.make_async_copy(
            stage_v[par].at[:, :, pl.ds(0, BT)], out_hbm.at[0, :, 0], wsems[par]
        ).wait()

    dt_idx = [(16 * q + iota16) // 8 for q in range(D // 16)]
    s_idx = [(16 * q + iota16) % 8 for q in range(D // 16)]

    def transpose(par):
        rows = rows_v[par]
        stage = stage_v[par]

        def c_step(cc, carry):
            c0 = cc * 8
            bls = [jnp.full((16,), 0, jnp.int32) + (c0 + j) for j in range(8)]
            vs = [
                rows[c0 + j, pl.ds(16 * q, 16)]
                for j in range(8)
                for q in range(D // 16)
            ]
            for j in range(8):
                for q in range(D // 16):
                    plsc.store_scatter(
                        stage, [dt_idx[q], s_idx[q], bls[j]], vs[j * (D // 16) + q]
                    )
            return carry

        lax.fori_loop(0, BT // 8, c_step, 0)

    for par in range(NBUF):
        idx_copy_async(par, par)
    for par in range(NBUF):
        wait_idx(par)
        start_gather(par)

    def body(gg, carry):
        for par in range(NBUF):
            wait_gather(par)

            @pl.when(gg < GROUPS - 1)
            def _():
                idx_copy_async(NBUF * gg + NBUF + par, par)

        for par in range(NBUF):
            l = NBUF * gg + par

            @pl.when(gg >= 1)
            def _():
                wait_write(par)

            transpose(par)
            start_write(l, par)

        for par in range(NBUF):
            @pl.when(gg < GROUPS - 1)
            def _():
                wait_idx(par)
                start_gather(par)

        return carry

    lax.fori_loop(0, GROUPS, body, 0)
    for par in range(NBUF):
        wait_write(par)


def _sc_gather(tokens_T, table):
    mesh = plsc.VectorSubcoreMesh(core_axis_name="c", subcore_axis_name="s")
    k = functools.partial(
        pl.kernel,
        mesh=mesh,
        out_type=jax.ShapeDtypeStruct((L, DT, NW, 8, BT), jnp.float32),
        scratch_types=[
            [pltpu.VMEM((BT,), jnp.int32) for _ in range(NBUF)],
            [pltpu.VMEM((BT, D), jnp.float32) for _ in range(NBUF)],
            [pltpu.VMEM((DT, 8, BT + 5), jnp.float32) for _ in range(NBUF)],
            [pltpu.SemaphoreType.DMA for _ in range(NBUF)],
            [pltpu.SemaphoreType.DMA for _ in range(NBUF)],
            [pltpu.SemaphoreType.DMA for _ in range(NBUF)],
        ],
        compiler_params=pltpu.CompilerParams(
            use_tc_tiling_on_sc=False, needs_layout_passes=False
        ),
    )(_gather_body)
    return k(tokens_T, table)


def _mask_body(tok_ref, pad_ref, seq_ref):
    pad_ref[...] = (tok_ref[...] == PAD).astype(jnp.int8)
    r = lax.broadcasted_iota(jnp.int32, (L, L), 0)
    c = lax.broadcasted_iota(jnp.int32, (L, L), 1)
    seq_ref[...] = (c > r).astype(jnp.int8)


def _tc_masks(tokens):
    return pl.pallas_call(
        _mask_body,
        out_shape=(
            jax.ShapeDtypeStruct((B, L), jnp.int8),
            jax.ShapeDtypeStruct((L, L), jnp.int8),
        ),
    )(tokens)


def kernel(tokens, table):
    arr = _sc_gather(tokens.T, table)
    features = arr.transpose(2, 4, 0, 1, 3).reshape(B, L, D)
    pad8, seq8 = _tc_masks(tokens)
    padding_masks = pad8.astype(bool)[:, None, None, :]
    sequential_masks = seq8.astype(bool)
    return (features, padding_masks, sequential_masks)
